# Initial kernel scaffold; baseline (speedup 1.0000x reference)
import jax
import jax.numpy as jnp
from jax.experimental import pallas as pl


def kernel(adj_mat_list, bw0, bc0, bw1, bc1):
    raise NotImplementedError("write your pallas kernel here")


# speedup vs baseline: 1.4794x; 1.4794x over previous
import jax
import jax.numpy as jnp
from jax.experimental import pallas as pl
from jax.experimental.pallas import tpu as pltpu

_BM = 256


def _layer0_body(bc0_ref, adj_ref, bw0_ref, out0_ref):
    nrel = adj_ref.shape[0]
    acc = jnp.zeros(out0_ref.shape, jnp.float32)
    for r in range(nrel):
        w = bc0_ref[r, 0] * bw0_ref[0] + bc0_ref[r, 1] * bw0_ref[1]
        acc += jnp.dot(adj_ref[r], w, preferred_element_type=jnp.float32)
    out0_ref[...] = acc


def _layer1_body(bc1_ref, adj_ref, out0_ref, bw1_ref, out1_ref, fsum_ref,
                 g_ref):
    i = pl.program_id(0)
    nrel = adj_ref.shape[0]

    @pl.when(i == 0)
    def _init():
        h = jnp.maximum(out0_ref[...], 0.0)
        for r in range(nrel):
            w = bc1_ref[r, 0] * bw1_ref[0] + bc1_ref[r, 1] * bw1_ref[1]
            g_ref[r] = jnp.dot(h, w, preferred_element_type=jnp.float32)
        fsum_ref[0:1, :] = jnp.sum(out0_ref[...], axis=0, keepdims=True)
        fsum_ref[1:2, :] = jnp.zeros((1, fsum_ref.shape[1]), jnp.float32)

    acc = jnp.zeros(out1_ref.shape, jnp.float32)
    for r in range(nrel):
        acc += jnp.dot(adj_ref[r], g_ref[r], preferred_element_type=jnp.float32)
    out1_ref[...] = acc
    fsum_ref[1:2, :] += jnp.sum(acc, axis=0, keepdims=True)


def kernel(adj_mat_list, bw0, bc0, bw1, bc1):
    nrel, n, _ = adj_mat_list.shape
    nb, _, h0 = bw0.shape
    h1 = bw1.shape[2]
    grid = (n // _BM,)

    out0 = pl.pallas_call(
        _layer0_body,
        grid=grid,
        in_specs=[
            pl.BlockSpec(memory_space=pltpu.SMEM),
            pl.BlockSpec((nrel, _BM, n), lambda i: (0, i, 0)),
            pl.BlockSpec((nb, n, h0), lambda i: (0, 0, 0)),
        ],
        out_specs=pl.BlockSpec((_BM, h0), lambda i: (i, 0)),
        out_shape=jax.ShapeDtypeStruct((n, h0), jnp.float32),
    )(bc0, adj_mat_list, bw0)

    out1, fsum = pl.pallas_call(
        _layer1_body,
        grid=grid,
        in_specs=[
            pl.BlockSpec(memory_space=pltpu.SMEM),
            pl.BlockSpec((nrel, _BM, n), lambda i: (0, i, 0)),
            pl.BlockSpec((n, h0), lambda i: (0, 0)),
            pl.BlockSpec((nb, h0, h1), lambda i: (0, 0, 0)),
        ],
        out_specs=[
            pl.BlockSpec((_BM, h1), lambda i: (i, 0)),
            pl.BlockSpec((2, h0), lambda i: (0, 0)),
        ],
        out_shape=[
            jax.ShapeDtypeStruct((n, h1), jnp.float32),
            jax.ShapeDtypeStruct((2, h0), jnp.float32),
        ],
        scratch_shapes=[pltpu.VMEM((nrel, n, h0), jnp.float32)],
    )(bc1, adj_mat_list, out0, bw1)

    final_rep = fsum.reshape(1, h0 + h1)
    return (out1, final_rep)


# speedup vs baseline: 1.5217x; 1.0286x over previous
import jax
import jax.numpy as jnp
from jax.experimental import pallas as pl
from jax.experimental.pallas import tpu as pltpu

_BM = 256


def _body(bc0_ref, bc1_ref, adj_ref, bw0_ref, bw1_ref, out1_ref, fsum_ref,
          out0_ref, g_ref):
    phase = pl.program_id(0)
    i = pl.program_id(1)
    nrel = adj_ref.shape[0]
    bm = adj_ref.shape[1]

    @pl.when(phase == 0)
    def _layer0():
        acc = jnp.zeros((bm, out0_ref.shape[1]), jnp.float32)
        for r in range(nrel):
            w = bc0_ref[r, 0] * bw0_ref[0] + bc0_ref[r, 1] * bw0_ref[1]
            acc += jnp.dot(adj_ref[r], w, preferred_element_type=jnp.float32)
        out0_ref[pl.ds(i * bm, bm), :] = acc

    @pl.when(jnp.logical_and(phase == 1, i == 0))
    def _between():
        h = jnp.maximum(out0_ref[...], 0.0)
        for r in range(nrel):
            w = bc1_ref[r, 0] * bw1_ref[0] + bc1_ref[r, 1] * bw1_ref[1]
            g_ref[r] = jnp.dot(h, w, preferred_element_type=jnp.float32)
        fsum_ref[0:1, :] = jnp.sum(out0_ref[...], axis=0, keepdims=True)
        fsum_ref[1:2, :] = jnp.zeros((1, fsum_ref.shape[1]), jnp.float32)

    @pl.when(phase == 1)
    def _layer1():
        acc = jnp.zeros(out1_ref.shape, jnp.float32)
        for r in range(nrel):
            acc += jnp.dot(adj_ref[r], g_ref[r],
                           preferred_element_type=jnp.float32)
        out1_ref[...] = acc
        fsum_ref[1:2, :] += jnp.sum(acc, axis=0, keepdims=True)


def kernel(adj_mat_list, bw0, bc0, bw1, bc1):
    nrel, n, _ = adj_mat_list.shape
    nb, _, h0 = bw0.shape
    h1 = bw1.shape[2]
    grid = (2, n // _BM)

    out1, fsum = pl.pallas_call(
        _body,
        grid=grid,
        in_specs=[
            pl.BlockSpec(memory_space=pltpu.SMEM),
            pl.BlockSpec(memory_space=pltpu.SMEM),
            pl.BlockSpec((nrel, _BM, n), lambda p, i: (0, i, 0)),
            pl.BlockSpec((nb, n, h0), lambda p, i: (0, 0, 0)),
            pl.BlockSpec((nb, h0, h1), lambda p, i: (0, 0, 0)),
        ],
        out_specs=[
            pl.BlockSpec((_BM, h1), lambda p, i: (i, 0)),
            pl.BlockSpec((2, h0), lambda p, i: (0, 0)),
        ],
        out_shape=[
            jax.ShapeDtypeStruct((n, h1), jnp.float32),
            jax.ShapeDtypeStruct((2, h0), jnp.float32),
        ],
        scratch_shapes=[
            pltpu.VMEM((n, h0), jnp.float32),
            pltpu.VMEM((nrel, n, h0), jnp.float32),
        ],
    )(bc0, bc1, adj_mat_list, bw0, bw1)

    final_rep = fsum.reshape(1, h0 + h1)
    return (out1, final_rep)
